# baseline (speedup 1.0000x reference)
import functools

import jax
import jax.numpy as jnp
from jax import lax
from jax.experimental import pallas as pl
from jax.experimental.pallas import tpu as pltpu
from jax.experimental.pallas import tpu_sc as plsc

N = 10000
E = 320000
D = 128
NC = 2
NS = 16
NW = NC * NS
EPW = E // NW
K = 40
NCHUNK = EPW // K
NB = 4
NR = NCHUNK
ZR = 1000
NZ = N // ZR


def _sc_agg_body(h_hbm, src_hbm, dst_hbm, zrows_hbm, agg_out,
                 srcv, dstv, rows, acc_sh, sem_g, sem_s, sem_is, sem_id):
    c = lax.axis_index("c")
    s = lax.axis_index("s")
    w = c * NS + s

    @pl.when(s < NZ)
    def _zero():
        pltpu.sync_copy(zrows_hbm.at[pl.ds(s * ZR, ZR)],
                        acc_sh.at[pl.ds(s * ZR, ZR)])

    plsc.subcore_barrier()

    def fire_sidx(r, p):
        pltpu.async_copy(src_hbm.at[w, r], srcv.at[p], sem_is.at[p])

    def wait_sidx(p):
        pltpu.make_async_copy(src_hbm.at[0, 0], srcv.at[p],
                              sem_is.at[p]).wait()

    def fire_didx(r, p):
        pltpu.async_copy(dst_hbm.at[w, r], dstv.at[p], sem_id.at[p])

    def wait_didx(p):
        pltpu.make_async_copy(dst_hbm.at[0, 0], dstv.at[p],
                              sem_id.at[p]).wait()

    def fire_gather(p):
        pltpu.async_copy(h_hbm.at[srcv.at[p]], rows.at[p], sem_g.at[p])

    def wait_gather(p):
        pltpu.make_async_copy(zrows_hbm.at[pl.ds(0, K)], rows.at[p],
                              sem_g.at[p]).wait()

    def fire_scatter(p):
        pltpu.async_copy(rows.at[p], acc_sh.at[dstv.at[p]],
                         sem_s.at[p], add=True)

    def wait_scatter(p):
        pltpu.make_async_copy(zrows_hbm.at[pl.ds(0, K)], rows.at[p],
                              sem_s.at[p]).wait()

    fire_sidx(0, 0)
    fire_sidx(1, 1)
    fire_didx(0, 0)
    wait_sidx(0)
    fire_gather(0)

    def body(r, carry):
        p = lax.rem(r, NB)
        pn = lax.rem(r + 1, NB)

        wait_gather(p)

        @pl.when(r + 1 >= NB)
        def _free_next_slot():
            wait_scatter(pn)

        wait_didx(p)
        fire_scatter(p)

        @pl.when(r + 1 < NR)
        def _next():
            wait_sidx(pn)
            fire_gather(pn)
            fire_didx(r + 1, pn)

            @pl.when(r + 2 < NR)
            def _next2():
                fire_sidx(r + 2, lax.rem(r + 2, NB))

        return carry

    lax.fori_loop(0, NR, body, 0, unroll=False)
    for d in range(1, NB):
        wait_scatter((NR - d) % NB)
    plsc.subcore_barrier()

    @pl.when(s < NZ)
    def _writeout():
        pltpu.sync_copy(acc_sh.at[pl.ds(s * ZR, ZR)],
                        agg_out.at[c, pl.ds(s * ZR, ZR)])


_sc_agg = pl.kernel(
    _sc_agg_body,
    out_type=jax.ShapeDtypeStruct((NC, N, D), jnp.float32),
    mesh=plsc.VectorSubcoreMesh(core_axis_name="c", subcore_axis_name="s",
                                num_cores=NC, num_subcores=NS),
    scratch_types=[
        pltpu.VMEM((NB, K), jnp.int32),
        pltpu.VMEM((NB, K), jnp.int32),
        pltpu.VMEM((NB, K, D), jnp.float32),
        pltpu.VMEM_SHARED((N, D), jnp.float32),
        pltpu.SemaphoreType.DMA((NB,)),
        pltpu.SemaphoreType.DMA((NB,)),
        pltpu.SemaphoreType.DMA((NB,)),
        pltpu.SemaphoreType.DMA((NB,)),
    ],
    name="sc_sage_agg",
)


def _sc_cnt_body(dst_hbm, zrows_hbm, ones_hbm, cnt_out,
                 dstv, ones_v, acc_sh, sem_s, sem_i):
    c = lax.axis_index("c")
    s = lax.axis_index("s")
    w = c * NS + s

    @pl.when(s < NZ)
    def _zero():
        pltpu.sync_copy(zrows_hbm.at[pl.ds(s * ZR, ZR)],
                        acc_sh.at[pl.ds(s * ZR, ZR)])

    pltpu.sync_copy(ones_hbm, ones_v)
    plsc.subcore_barrier()

    def fire_idx(r, p):
        pltpu.async_copy(dst_hbm.at[w, r], dstv.at[p], sem_i.at[p])

    def wait_idx(p):
        pltpu.make_async_copy(dst_hbm.at[0, 0], dstv.at[p],
                              sem_i.at[p]).wait()

    def fire_scatter(p):
        pltpu.async_copy(ones_v, acc_sh.at[dstv.at[p]],
                         sem_s.at[p], add=True)

    def wait_scatter(p):
        pltpu.make_async_copy(zrows_hbm.at[pl.ds(0, K)], ones_v,
                              sem_s.at[p]).wait()

    fire_idx(0, 0)
    fire_idx(1, 1)

    def body(r, carry):
        p = lax.rem(r, NB)
        p2 = lax.rem(r + 2, NB)

        wait_idx(p)

        @pl.when(r + 2 >= NB)
        def _free_slot2():
            wait_scatter(p2)

        fire_scatter(p)

        @pl.when(r + 2 < NR)
        def _next():
            fire_idx(r + 2, p2)

        return carry

    lax.fori_loop(0, NR, body, 0, unroll=False)
    for d in range(1, NB - 1):
        wait_scatter((NR - d) % NB)
    plsc.subcore_barrier()

    @pl.when(s < NZ)
    def _writeout():
        pltpu.sync_copy(acc_sh.at[pl.ds(s * ZR, ZR)],
                        cnt_out.at[c, pl.ds(s * ZR, ZR)])


_sc_cnt = pl.kernel(
    _sc_cnt_body,
    out_type=jax.ShapeDtypeStruct((NC, N, D), jnp.float32),
    mesh=plsc.VectorSubcoreMesh(core_axis_name="c", subcore_axis_name="s",
                                num_cores=NC, num_subcores=NS),
    scratch_types=[
        pltpu.VMEM((NB, K), jnp.int32),
        pltpu.VMEM((K, D), jnp.float32),
        pltpu.VMEM_SHARED((N, D), jnp.float32),
        pltpu.SemaphoreType.DMA((NB,)),
        pltpu.SemaphoreType.DMA((NB,)),
    ],
    name="sc_sage_cnt",
)


def _tc_body(do_relu, aggp_ref, cntp_ref, h_ref, wl_ref, bl_ref, wr_ref,
             out_ref):
    agg = aggp_ref[0] + aggp_ref[1]
    cnt = cntp_ref[0] + cntp_ref[1]
    mean = agg / jnp.maximum(cnt, 1.0)
    dn = (((1,), (1,)), ((), ()))
    y = (lax.dot_general(mean, wl_ref[...], dn,
                         preferred_element_type=jnp.float32)
         + lax.dot_general(h_ref[...], wr_ref[...], dn,
                           preferred_element_type=jnp.float32)
         + bl_ref[...])
    out_ref[...] = jnp.maximum(y, 0.0) if do_relu else y


def _tc_layer(aggp, cntp, h, wl, bl, wr, do_relu):
    R = 400
    grid = (N // R,)
    return pl.pallas_call(
        functools.partial(_tc_body, do_relu),
        grid=grid,
        in_specs=[
            pl.BlockSpec((NC, R, D), lambda i: (0, i, 0)),
            pl.BlockSpec((NC, R, D), lambda i: (0, i, 0)),
            pl.BlockSpec((R, D), lambda i: (i, 0)),
            pl.BlockSpec((D, D), lambda i: (0, 0)),
            pl.BlockSpec((1, D), lambda i: (0, 0)),
            pl.BlockSpec((D, D), lambda i: (0, 0)),
        ],
        out_specs=pl.BlockSpec((R, D), lambda i: (i, 0)),
        out_shape=jax.ShapeDtypeStruct((N, D), jnp.float32),
    )(aggp, cntp, h, wl, bl.reshape(1, D), wr)


def kernel(x, edge_index, Wl1, bl1, Wr1, Wl2, bl2, Wr2, Wl3, bl3, Wr3):
    src = edge_index[0].reshape(NW, NCHUNK, K)
    dst = edge_index[1].reshape(NW, NCHUNK, K)
    zrows = jnp.zeros((N, D), jnp.float32)
    ones = jnp.ones((K, D), jnp.float32)

    cntp = _sc_cnt(dst, zrows, ones)
    aggp = _sc_agg(x, src, dst, zrows)
    h1 = _tc_layer(aggp, cntp, x, Wl1, bl1, Wr1, True)
    aggp = _sc_agg(h1, src, dst, zrows)
    h2 = _tc_layer(aggp, cntp, h1, Wl2, bl2, Wr2, True)
    aggp = _sc_agg(h2, src, dst, zrows)
    return _tc_layer(aggp, cntp, h2, Wl3, bl3, Wr3, False)


# speedup vs baseline: 1.3291x; 1.3291x over previous
import functools

import jax
import jax.numpy as jnp
from jax import lax
from jax.experimental import pallas as pl
from jax.experimental.pallas import tpu as pltpu
from jax.experimental.pallas import tpu_sc as plsc

N = 10000
E = 320000
D = 128
NC = 2
NS = 16
NW = NC * NS
EPW = E // NW
K = 40
NCHUNK = EPW // K
G = 2
NR = NCHUNK // G
ZR = 1000
NZ = N // ZR


def _sc_agg_body(h_hbm, src_hbm, dst_hbm, zrows_hbm, agg_out,
                 srcv, dstv, rows, acc_sh, sem_g, sem_s, sem_is, sem_id):
    c = lax.axis_index("c")
    s = lax.axis_index("s")
    w = c * NS + s

    @pl.when(s < NZ)
    def _zero():
        pltpu.sync_copy(zrows_hbm.at[pl.ds(s * ZR, ZR)],
                        acc_sh.at[pl.ds(s * ZR, ZR)])

    plsc.subcore_barrier()

    def fire_sidx(r, p):
        for i in range(G):
            pltpu.async_copy(src_hbm.at[w, r * G + i],
                             srcv.at[p * G + i], sem_is.at[p])

    def wait_sidx(p):
        for i in range(G):
            pltpu.make_async_copy(src_hbm.at[0, 0],
                                  srcv.at[p * G + i], sem_is.at[p]).wait()

    def fire_didx(r, p):
        for i in range(G):
            pltpu.async_copy(dst_hbm.at[w, r * G + i],
                             dstv.at[p * G + i], sem_id.at[p])

    def wait_didx(p):
        for i in range(G):
            pltpu.make_async_copy(dst_hbm.at[0, 0],
                                  dstv.at[p * G + i], sem_id.at[p]).wait()

    def fire_gather(p):
        for i in range(G):
            pltpu.async_copy(h_hbm.at[srcv.at[p * G + i]],
                             rows.at[p, i], sem_g.at[p])

    def wait_gather(p):
        for i in range(G):
            pltpu.make_async_copy(zrows_hbm.at[pl.ds(0, K)],
                                  rows.at[p, i], sem_g.at[p]).wait()

    def fire_scatter(p):
        for i in range(G):
            pltpu.async_copy(rows.at[p, i], acc_sh.at[dstv.at[p * G + i]],
                             sem_s.at[p], add=True)

    def wait_scatter(p):
        for i in range(G):
            pltpu.make_async_copy(zrows_hbm.at[pl.ds(0, K)],
                                  rows.at[p, i], sem_s.at[p]).wait()

    fire_sidx(0, 0)
    fire_sidx(1, 1)
    fire_didx(0, 0)
    wait_sidx(0)
    fire_gather(0)

    def body(r, carry):
        p = lax.rem(r, 2)
        q = 1 - p
        wait_gather(p)
        wait_didx(p)
        fire_scatter(p)

        @pl.when(r > 0)
        def _drain_prev():
            wait_scatter(q)

        @pl.when(r + 1 < NR)
        def _next():
            wait_sidx(q)
            fire_gather(q)
            fire_didx(r + 1, q)

            @pl.when(r + 2 < NR)
            def _next2():
                fire_sidx(r + 2, p)

        return carry

    lax.fori_loop(0, NR, body, 0, unroll=False)
    wait_scatter((NR - 1) % 2)
    plsc.subcore_barrier()

    @pl.when(s < NZ)
    def _writeout():
        pltpu.sync_copy(acc_sh.at[pl.ds(s * ZR, ZR)],
                        agg_out.at[c, pl.ds(s * ZR, ZR)])


_sc_agg = pl.kernel(
    _sc_agg_body,
    out_type=jax.ShapeDtypeStruct((NC, N, D), jnp.float32),
    mesh=plsc.VectorSubcoreMesh(core_axis_name="c", subcore_axis_name="s",
                                num_cores=NC, num_subcores=NS),
    scratch_types=[
        pltpu.VMEM((2 * G, K), jnp.int32),
        pltpu.VMEM((2 * G, K), jnp.int32),
        pltpu.VMEM((2, G, K, D), jnp.float32),
        pltpu.VMEM_SHARED((N, D), jnp.float32),
        pltpu.SemaphoreType.DMA((2,)),
        pltpu.SemaphoreType.DMA((2,)),
        pltpu.SemaphoreType.DMA((2,)),
        pltpu.SemaphoreType.DMA((2,)),
    ],
    name="sc_sage_agg",
)


def _sc_cnt_body(dst_hbm, zrows_hbm, ones_hbm, cnt_out,
                 dstv, ones_v, acc_sh, sem_s, sem_i):
    c = lax.axis_index("c")
    s = lax.axis_index("s")
    w = c * NS + s

    @pl.when(s < NZ)
    def _zero():
        pltpu.sync_copy(zrows_hbm.at[pl.ds(s * ZR, ZR)],
                        acc_sh.at[pl.ds(s * ZR, ZR)])

    pltpu.sync_copy(ones_hbm, ones_v)
    plsc.subcore_barrier()

    def fire_idx(r, p):
        for i in range(G):
            pltpu.async_copy(dst_hbm.at[w, r * G + i],
                             dstv.at[p * G + i], sem_i.at[p])

    def wait_idx(p):
        for i in range(G):
            pltpu.make_async_copy(dst_hbm.at[0, 0],
                                  dstv.at[p * G + i], sem_i.at[p]).wait()

    def fire_scatter(p):
        for i in range(G):
            pltpu.async_copy(ones_v, acc_sh.at[dstv.at[p * G + i]],
                             sem_s.at[p], add=True)

    def wait_scatter(p):
        for i in range(G):
            pltpu.make_async_copy(zrows_hbm.at[pl.ds(0, K)], ones_v,
                                  sem_s.at[p]).wait()

    fire_idx(0, 0)

    def body(r, carry):
        p = lax.rem(r, 2)
        q = 1 - p
        wait_idx(p)
        fire_scatter(p)

        @pl.when(r > 0)
        def _drain_prev():
            wait_scatter(q)

        @pl.when(r + 1 < NR)
        def _next():
            fire_idx(r + 1, q)

        return carry

    lax.fori_loop(0, NR, body, 0, unroll=False)
    wait_scatter((NR - 1) % 2)
    plsc.subcore_barrier()

    @pl.when(s < NZ)
    def _writeout():
        pltpu.sync_copy(acc_sh.at[pl.ds(s * ZR, ZR)],
                        cnt_out.at[c, pl.ds(s * ZR, ZR)])


_sc_cnt = pl.kernel(
    _sc_cnt_body,
    out_type=jax.ShapeDtypeStruct((NC, N, D), jnp.float32),
    mesh=plsc.VectorSubcoreMesh(core_axis_name="c", subcore_axis_name="s",
                                num_cores=NC, num_subcores=NS),
    scratch_types=[
        pltpu.VMEM((2 * G, K), jnp.int32),
        pltpu.VMEM((K, D), jnp.float32),
        pltpu.VMEM_SHARED((N, D), jnp.float32),
        pltpu.SemaphoreType.DMA((2,)),
        pltpu.SemaphoreType.DMA((2,)),
    ],
    name="sc_sage_cnt",
)


def _tc_body(do_relu, aggp_ref, cntp_ref, h_ref, wl_ref, bl_ref, wr_ref,
             out_ref):
    agg = aggp_ref[0] + aggp_ref[1]
    cnt = cntp_ref[0] + cntp_ref[1]
    mean = agg / jnp.maximum(cnt, 1.0)
    dn = (((1,), (1,)), ((), ()))
    y = (lax.dot_general(mean, wl_ref[...], dn,
                         preferred_element_type=jnp.float32)
         + lax.dot_general(h_ref[...], wr_ref[...], dn,
                           preferred_element_type=jnp.float32)
         + bl_ref[...])
    out_ref[...] = jnp.maximum(y, 0.0) if do_relu else y


def _tc_layer(aggp, cntp, h, wl, bl, wr, do_relu):
    R = 400
    grid = (N // R,)
    return pl.pallas_call(
        functools.partial(_tc_body, do_relu),
        grid=grid,
        in_specs=[
            pl.BlockSpec((NC, R, D), lambda i: (0, i, 0)),
            pl.BlockSpec((NC, R, D), lambda i: (0, i, 0)),
            pl.BlockSpec((R, D), lambda i: (i, 0)),
            pl.BlockSpec((D, D), lambda i: (0, 0)),
            pl.BlockSpec((1, D), lambda i: (0, 0)),
            pl.BlockSpec((D, D), lambda i: (0, 0)),
        ],
        out_specs=pl.BlockSpec((R, D), lambda i: (i, 0)),
        out_shape=jax.ShapeDtypeStruct((N, D), jnp.float32),
    )(aggp, cntp, h, wl, bl.reshape(1, D), wr)


def kernel(x, edge_index, Wl1, bl1, Wr1, Wl2, bl2, Wr2, Wl3, bl3, Wr3):
    src = edge_index[0].reshape(NW, NCHUNK, K)
    dst = edge_index[1].reshape(NW, NCHUNK, K)
    zrows = jnp.zeros((N, D), jnp.float32)
    ones = jnp.ones((K, D), jnp.float32)

    cntp = _sc_cnt(dst, zrows, ones)
    aggp = _sc_agg(x, src, dst, zrows)
    h1 = _tc_layer(aggp, cntp, x, Wl1, bl1, Wr1, True)
    aggp = _sc_agg(h1, src, dst, zrows)
    h2 = _tc_layer(aggp, cntp, h1, Wl2, bl2, Wr2, True)
    aggp = _sc_agg(h2, src, dst, zrows)
    return _tc_layer(aggp, cntp, h2, Wl3, bl3, Wr3, False)


# speedup vs baseline: 1.3764x; 1.0356x over previous
import functools

import jax
import jax.numpy as jnp
from jax import lax
from jax.experimental import pallas as pl
from jax.experimental.pallas import tpu as pltpu
from jax.experimental.pallas import tpu_sc as plsc

N = 10000
E = 320000
D = 128
NC = 2
NS = 16
NW = NC * NS
EPW = E // NW
K = 40
NCHUNK = EPW // K
G = 2
NR = NCHUNK // G
ZR = 1000
NZ = N // ZR


def _sc_agg_body(with_cnt, h_hbm, src_hbm, dst_hbm, zrows_hbm, ones_hbm,
                 agg_out, cnt_out, srcv, dstv, rows, ones_v, acc_sh,
                 sem_g, sem_s, sem_is, sem_id, sem_c):
    c = lax.axis_index("c")
    s = lax.axis_index("s")
    w = c * NS + s

    def zero_acc():
        @pl.when(s < NZ)
        def _zero():
            pltpu.sync_copy(zrows_hbm.at[pl.ds(s * ZR, ZR)],
                            acc_sh.at[pl.ds(s * ZR, ZR)])

    zero_acc()
    if with_cnt:
        pltpu.sync_copy(ones_hbm, ones_v)
        plsc.subcore_barrier()

        def cfire_idx(r, p):
            for i in range(G):
                pltpu.async_copy(dst_hbm.at[w, r * G + i],
                                 dstv.at[p * G + i], sem_id.at[p])

        def cwait_idx(p):
            for i in range(G):
                pltpu.make_async_copy(dst_hbm.at[0, 0],
                                      dstv.at[p * G + i], sem_id.at[p]).wait()

        def cfire_scatter(p):
            for i in range(G):
                pltpu.async_copy(ones_v, acc_sh.at[dstv.at[p * G + i]],
                                 sem_c.at[p], add=True)

        def cwait_scatter(p):
            for i in range(G):
                pltpu.make_async_copy(zrows_hbm.at[pl.ds(0, K)], ones_v,
                                      sem_c.at[p]).wait()

        cfire_idx(0, 0)

        def cbody(r, carry):
            p = lax.rem(r, 2)
            q = 1 - p
            cwait_idx(p)
            cfire_scatter(p)

            @pl.when(r > 0)
            def _drain_prev():
                cwait_scatter(q)

            @pl.when(r + 1 < NR)
            def _next():
                cfire_idx(r + 1, q)

            return carry

        lax.fori_loop(0, NR, cbody, 0, unroll=False)
        cwait_scatter((NR - 1) % 2)
        plsc.subcore_barrier()

        @pl.when(s < NZ)
        def _cnt_writeout():
            pltpu.sync_copy(acc_sh.at[pl.ds(s * ZR, ZR)],
                            cnt_out.at[c, pl.ds(s * ZR, ZR)])

        plsc.subcore_barrier()
        zero_acc()
    plsc.subcore_barrier()

    def fire_sidx(r, p):
        for i in range(G):
            pltpu.async_copy(src_hbm.at[w, r * G + i],
                             srcv.at[p * G + i], sem_is.at[p])

    def wait_sidx(p):
        for i in range(G):
            pltpu.make_async_copy(src_hbm.at[0, 0],
                                  srcv.at[p * G + i], sem_is.at[p]).wait()

    def fire_didx(r, p):
        for i in range(G):
            pltpu.async_copy(dst_hbm.at[w, r * G + i],
                             dstv.at[p * G + i], sem_id.at[p])

    def wait_didx(p):
        for i in range(G):
            pltpu.make_async_copy(dst_hbm.at[0, 0],
                                  dstv.at[p * G + i], sem_id.at[p]).wait()

    def fire_gather(p):
        for i in range(G):
            pltpu.async_copy(h_hbm.at[srcv.at[p * G + i]],
                             rows.at[p, i], sem_g.at[p])

    def wait_gather(p):
        for i in range(G):
            pltpu.make_async_copy(zrows_hbm.at[pl.ds(0, K)],
                                  rows.at[p, i], sem_g.at[p]).wait()

    def fire_scatter(p):
        for i in range(G):
            pltpu.async_copy(rows.at[p, i], acc_sh.at[dstv.at[p * G + i]],
                             sem_s.at[p], add=True)

    def wait_scatter(p):
        for i in range(G):
            pltpu.make_async_copy(zrows_hbm.at[pl.ds(0, K)],
                                  rows.at[p, i], sem_s.at[p]).wait()

    fire_sidx(0, 0)
    fire_sidx(1, 1)
    fire_didx(0, 0)
    wait_sidx(0)
    fire_gather(0)

    def body(r, carry):
        p = lax.rem(r, 2)
        q = 1 - p
        wait_gather(p)
        wait_didx(p)
        fire_scatter(p)

        @pl.when(r > 0)
        def _drain_prev():
            wait_scatter(q)

        @pl.when(r + 1 < NR)
        def _next():
            wait_sidx(q)
            fire_gather(q)
            fire_didx(r + 1, q)

            @pl.when(r + 2 < NR)
            def _next2():
                fire_sidx(r + 2, p)

        return carry

    lax.fori_loop(0, NR, body, 0, unroll=False)
    wait_scatter((NR - 1) % 2)
    plsc.subcore_barrier()

    @pl.when(s < NZ)
    def _writeout():
        pltpu.sync_copy(acc_sh.at[pl.ds(s * ZR, ZR)],
                        agg_out.at[c, pl.ds(s * ZR, ZR)])


def _make_sc_agg(with_cnt):
    outs = jax.ShapeDtypeStruct((NC, N, D), jnp.float32)
    return pl.kernel(
        functools.partial(_sc_agg_body, with_cnt),
        out_type=(outs, outs) if with_cnt else (outs, outs),
        mesh=plsc.VectorSubcoreMesh(core_axis_name="c", subcore_axis_name="s",
                                    num_cores=NC, num_subcores=NS),
        scratch_types=[
            pltpu.VMEM((2 * G, K), jnp.int32),
            pltpu.VMEM((2 * G, K), jnp.int32),
            pltpu.VMEM((2, G, K, D), jnp.float32),
            pltpu.VMEM((K, D), jnp.float32),
            pltpu.VMEM_SHARED((N, D), jnp.float32),
            pltpu.SemaphoreType.DMA((2,)),
            pltpu.SemaphoreType.DMA((2,)),
            pltpu.SemaphoreType.DMA((2,)),
            pltpu.SemaphoreType.DMA((2,)),
            pltpu.SemaphoreType.DMA((2,)),
        ],
        name="sc_sage_agg_cnt" if with_cnt else "sc_sage_agg",
    )


_sc_agg_cnt = _make_sc_agg(True)
_sc_agg = _make_sc_agg(False)


def _tc_body(do_relu, aggp_ref, cntp_ref, h_ref, wl_ref, bl_ref, wr_ref,
             out_ref):
    agg = aggp_ref[0] + aggp_ref[1]
    cnt = cntp_ref[0] + cntp_ref[1]
    mean = agg / jnp.maximum(cnt, 1.0)
    dn = (((1,), (1,)), ((), ()))
    y = (lax.dot_general(mean, wl_ref[...], dn,
                         preferred_element_type=jnp.float32)
         + lax.dot_general(h_ref[...], wr_ref[...], dn,
                           preferred_element_type=jnp.float32)
         + bl_ref[...])
    out_ref[...] = jnp.maximum(y, 0.0) if do_relu else y


def _tc_layer(aggp, cntp, h, wl, bl, wr, do_relu):
    R = 1000
    grid = (N // R,)
    return pl.pallas_call(
        functools.partial(_tc_body, do_relu),
        grid=grid,
        in_specs=[
            pl.BlockSpec((NC, R, D), lambda i: (0, i, 0)),
            pl.BlockSpec((NC, R, D), lambda i: (0, i, 0)),
            pl.BlockSpec((R, D), lambda i: (i, 0)),
            pl.BlockSpec((D, D), lambda i: (0, 0)),
            pl.BlockSpec((1, D), lambda i: (0, 0)),
            pl.BlockSpec((D, D), lambda i: (0, 0)),
        ],
        out_specs=pl.BlockSpec((R, D), lambda i: (i, 0)),
        out_shape=jax.ShapeDtypeStruct((N, D), jnp.float32),
    )(aggp, cntp, h, wl, bl.reshape(1, D), wr)


def kernel(x, edge_index, Wl1, bl1, Wr1, Wl2, bl2, Wr2, Wl3, bl3, Wr3):
    src = edge_index[0].reshape(NW, NCHUNK, K)
    dst = edge_index[1].reshape(NW, NCHUNK, K)
    zrows = jnp.zeros((N, D), jnp.float32)
    ones = jnp.ones((K, D), jnp.float32)

    aggp, cntp = _sc_agg_cnt(x, src, dst, zrows, ones)
    h1 = _tc_layer(aggp, cntp, x, Wl1, bl1, Wr1, True)
    aggp, _ = _sc_agg(h1, src, dst, zrows, ones)
    h2 = _tc_layer(aggp, cntp, h1, Wl2, bl2, Wr2, True)
    aggp, _ = _sc_agg(h2, src, dst, zrows, ones)
    return _tc_layer(aggp, cntp, h2, Wl3, bl3, Wr3, False)


# speedup vs baseline: 1.4073x; 1.0224x over previous
import functools

import jax
import jax.numpy as jnp
from jax import lax
from jax.experimental import pallas as pl
from jax.experimental.pallas import tpu as pltpu
from jax.experimental.pallas import tpu_sc as plsc

N = 10000
E = 320000
D = 128
NC = 2
NS = 16
NW = NC * NS
EPW = E // NW
K = 80
NCHUNK = EPW // K
G = 1
NR = NCHUNK // G
ZR = 1000
NZ = N // ZR


def _sc_agg_body(with_cnt, h_hbm, src_hbm, dst_hbm, zrows_hbm, ones_hbm,
                 agg_out, cnt_out, srcv, dstv, rows, ones_v, acc_sh,
                 sem_g, sem_s, sem_is, sem_id, sem_c):
    c = lax.axis_index("c")
    s = lax.axis_index("s")
    w = c * NS + s

    def zero_acc():
        @pl.when(s < NZ)
        def _zero():
            pltpu.sync_copy(zrows_hbm.at[pl.ds(s * ZR, ZR)],
                            acc_sh.at[pl.ds(s * ZR, ZR)])

    zero_acc()
    if with_cnt:
        pltpu.sync_copy(ones_hbm, ones_v)
        plsc.subcore_barrier()

        def cfire_idx(r, p):
            for i in range(G):
                pltpu.async_copy(dst_hbm.at[w, r * G + i],
                                 dstv.at[p * G + i], sem_id.at[p])

        def cwait_idx(p):
            for i in range(G):
                pltpu.make_async_copy(dst_hbm.at[0, 0],
                                      dstv.at[p * G + i], sem_id.at[p]).wait()

        def cfire_scatter(p):
            for i in range(G):
                pltpu.async_copy(ones_v, acc_sh.at[dstv.at[p * G + i]],
                                 sem_c.at[p], add=True)

        def cwait_scatter(p):
            for i in range(G):
                pltpu.make_async_copy(zrows_hbm.at[pl.ds(0, K)], ones_v,
                                      sem_c.at[p]).wait()

        cfire_idx(0, 0)

        def cbody(r, carry):
            p = lax.rem(r, 2)
            q = 1 - p
            cwait_idx(p)
            cfire_scatter(p)

            @pl.when(r > 0)
            def _drain_prev():
                cwait_scatter(q)

            @pl.when(r + 1 < NR)
            def _next():
                cfire_idx(r + 1, q)

            return carry

        lax.fori_loop(0, NR, cbody, 0, unroll=False)
        cwait_scatter((NR - 1) % 2)
        plsc.subcore_barrier()

        @pl.when(s < NZ)
        def _cnt_writeout():
            pltpu.sync_copy(acc_sh.at[pl.ds(s * ZR, ZR)],
                            cnt_out.at[c, pl.ds(s * ZR, ZR)])

        plsc.subcore_barrier()
        zero_acc()
    plsc.subcore_barrier()

    def fire_sidx(r, p):
        for i in range(G):
            pltpu.async_copy(src_hbm.at[w, r * G + i],
                             srcv.at[p * G + i], sem_is.at[p])

    def wait_sidx(p):
        for i in range(G):
            pltpu.make_async_copy(src_hbm.at[0, 0],
                                  srcv.at[p * G + i], sem_is.at[p]).wait()

    def fire_didx(r, p):
        for i in range(G):
            pltpu.async_copy(dst_hbm.at[w, r * G + i],
                             dstv.at[p * G + i], sem_id.at[p])

    def wait_didx(p):
        for i in range(G):
            pltpu.make_async_copy(dst_hbm.at[0, 0],
                                  dstv.at[p * G + i], sem_id.at[p]).wait()

    def fire_gather(p):
        for i in range(G):
            pltpu.async_copy(h_hbm.at[srcv.at[p * G + i]],
                             rows.at[p, i], sem_g.at[p])

    def wait_gather(p):
        for i in range(G):
            pltpu.make_async_copy(zrows_hbm.at[pl.ds(0, K)],
                                  rows.at[p, i], sem_g.at[p]).wait()

    def fire_scatter(p):
        for i in range(G):
            pltpu.async_copy(rows.at[p, i], acc_sh.at[dstv.at[p * G + i]],
                             sem_s.at[p], add=True)

    def wait_scatter(p):
        for i in range(G):
            pltpu.make_async_copy(zrows_hbm.at[pl.ds(0, K)],
                                  rows.at[p, i], sem_s.at[p]).wait()

    fire_sidx(0, 0)
    fire_sidx(1, 1)
    fire_didx(0, 0)
    wait_sidx(0)
    fire_gather(0)

    def body(r, carry):
        p = lax.rem(r, 2)
        q = 1 - p
        wait_gather(p)
        wait_didx(p)
        fire_scatter(p)

        @pl.when(r > 0)
        def _drain_prev():
            wait_scatter(q)

        @pl.when(r + 1 < NR)
        def _next():
            wait_sidx(q)
            fire_gather(q)
            fire_didx(r + 1, q)

            @pl.when(r + 2 < NR)
            def _next2():
                fire_sidx(r + 2, p)

        return carry

    lax.fori_loop(0, NR, body, 0, unroll=False)
    wait_scatter((NR - 1) % 2)
    plsc.subcore_barrier()

    @pl.when(s < NZ)
    def _writeout():
        pltpu.sync_copy(acc_sh.at[pl.ds(s * ZR, ZR)],
                        agg_out.at[c, pl.ds(s * ZR, ZR)])


def _make_sc_agg(with_cnt):
    outs = jax.ShapeDtypeStruct((NC, N, D), jnp.float32)
    return pl.kernel(
        functools.partial(_sc_agg_body, with_cnt),
        out_type=(outs, outs) if with_cnt else (outs, outs),
        mesh=plsc.VectorSubcoreMesh(core_axis_name="c", subcore_axis_name="s",
                                    num_cores=NC, num_subcores=NS),
        scratch_types=[
            pltpu.VMEM((2 * G, K), jnp.int32),
            pltpu.VMEM((2 * G, K), jnp.int32),
            pltpu.VMEM((2, G, K, D), jnp.float32),
            pltpu.VMEM((K, D), jnp.float32),
            pltpu.VMEM_SHARED((N, D), jnp.float32),
            pltpu.SemaphoreType.DMA((2,)),
            pltpu.SemaphoreType.DMA((2,)),
            pltpu.SemaphoreType.DMA((2,)),
            pltpu.SemaphoreType.DMA((2,)),
            pltpu.SemaphoreType.DMA((2,)),
        ],
        name="sc_sage_agg_cnt" if with_cnt else "sc_sage_agg",
    )


_sc_agg_cnt = _make_sc_agg(True)
_sc_agg = _make_sc_agg(False)


def _tc_body(do_relu, aggp_ref, cntp_ref, h_ref, wl_ref, bl_ref, wr_ref,
             out_ref):
    agg = aggp_ref[0] + aggp_ref[1]
    cnt = cntp_ref[0] + cntp_ref[1]
    mean = agg / jnp.maximum(cnt, 1.0)
    dn = (((1,), (1,)), ((), ()))
    y = (lax.dot_general(mean, wl_ref[...], dn,
                         preferred_element_type=jnp.float32)
         + lax.dot_general(h_ref[...], wr_ref[...], dn,
                           preferred_element_type=jnp.float32)
         + bl_ref[...])
    out_ref[...] = jnp.maximum(y, 0.0) if do_relu else y


def _tc_layer(aggp, cntp, h, wl, bl, wr, do_relu):
    R = 1000
    grid = (N // R,)
    return pl.pallas_call(
        functools.partial(_tc_body, do_relu),
        grid=grid,
        in_specs=[
            pl.BlockSpec((NC, R, D), lambda i: (0, i, 0)),
            pl.BlockSpec((NC, R, D), lambda i: (0, i, 0)),
            pl.BlockSpec((R, D), lambda i: (i, 0)),
            pl.BlockSpec((D, D), lambda i: (0, 0)),
            pl.BlockSpec((1, D), lambda i: (0, 0)),
            pl.BlockSpec((D, D), lambda i: (0, 0)),
        ],
        out_specs=pl.BlockSpec((R, D), lambda i: (i, 0)),
        out_shape=jax.ShapeDtypeStruct((N, D), jnp.float32),
    )(aggp, cntp, h, wl, bl.reshape(1, D), wr)


def kernel(x, edge_index, Wl1, bl1, Wr1, Wl2, bl2, Wr2, Wl3, bl3, Wr3):
    src = edge_index[0].reshape(NW, NCHUNK, K)
    dst = edge_index[1].reshape(NW, NCHUNK, K)
    zrows = jnp.zeros((N, D), jnp.float32)
    ones = jnp.ones((K, D), jnp.float32)

    aggp, cntp = _sc_agg_cnt(x, src, dst, zrows, ones)
    h1 = _tc_layer(aggp, cntp, x, Wl1, bl1, Wr1, True)
    aggp, _ = _sc_agg(h1, src, dst, zrows, ones)
    h2 = _tc_layer(aggp, cntp, h1, Wl2, bl2, Wr2, True)
    aggp, _ = _sc_agg(h2, src, dst, zrows, ones)
    return _tc_layer(aggp, cntp, h2, Wl3, bl3, Wr3, False)


# speedup vs baseline: 1.4087x; 1.0010x over previous
import functools

import jax
import jax.numpy as jnp
from jax import lax
from jax.experimental import pallas as pl
from jax.experimental.pallas import tpu as pltpu
from jax.experimental.pallas import tpu_sc as plsc

N = 10000
E = 320000
D = 128
NC = 2
NS = 16
NW = NC * NS
EPW = E // NW
K = 80
NCHUNK = EPW // K
G = 1
NB = 3
NR = NCHUNK // G
ZR = 1000
NZ = N // ZR


def _sc_agg_body(with_cnt, h_hbm, src_hbm, dst_hbm, zrows_hbm, ones_hbm,
                 agg_out, cnt_out, srcv, dstv, rows, acc_sh,
                 sem_g, sem_s, sem_is, sem_id, sem_c):
    ones_v = rows.at[0, 0]
    c = lax.axis_index("c")
    s = lax.axis_index("s")
    w = c * NS + s

    def zero_acc():
        @pl.when(s < NZ)
        def _zero():
            pltpu.sync_copy(zrows_hbm.at[pl.ds(s * ZR, ZR)],
                            acc_sh.at[pl.ds(s * ZR, ZR)])

    zero_acc()
    if with_cnt:
        pltpu.sync_copy(ones_hbm, ones_v)
        plsc.subcore_barrier()

        def cfire_idx(r, p):
            for i in range(G):
                pltpu.async_copy(dst_hbm.at[w, r * G + i],
                                 dstv.at[p * G + i], sem_id.at[p])

        def cwait_idx(p):
            for i in range(G):
                pltpu.make_async_copy(dst_hbm.at[0, 0],
                                      dstv.at[p * G + i], sem_id.at[p]).wait()

        def cfire_scatter(p):
            for i in range(G):
                pltpu.async_copy(ones_v, acc_sh.at[dstv.at[p * G + i]],
                                 sem_c.at[p], add=True)

        def cwait_scatter(p):
            for i in range(G):
                pltpu.make_async_copy(zrows_hbm.at[pl.ds(0, K)], ones_v,
                                      sem_c.at[p]).wait()

        cfire_idx(0, 0)

        def cbody(r, carry):
            p = lax.rem(r, 2)
            q = 1 - p
            cwait_idx(p)
            cfire_scatter(p)

            @pl.when(r > 0)
            def _drain_prev():
                cwait_scatter(q)

            @pl.when(r + 1 < NR)
            def _next():
                cfire_idx(r + 1, q)

            return carry

        lax.fori_loop(0, NR, cbody, 0, unroll=False)
        cwait_scatter((NR - 1) % 2)
        plsc.subcore_barrier()

        @pl.when(s < NZ)
        def _cnt_writeout():
            pltpu.sync_copy(acc_sh.at[pl.ds(s * ZR, ZR)],
                            cnt_out.at[c, pl.ds(s * ZR, ZR)])

        plsc.subcore_barrier()
        zero_acc()
    plsc.subcore_barrier()

    def fire_sidx(r, p):
        for i in range(G):
            pltpu.async_copy(src_hbm.at[w, r * G + i],
                             srcv.at[p * G + i], sem_is.at[p])

    def wait_sidx(p):
        for i in range(G):
            pltpu.make_async_copy(src_hbm.at[0, 0],
                                  srcv.at[p * G + i], sem_is.at[p]).wait()

    def fire_didx(r, p):
        for i in range(G):
            pltpu.async_copy(dst_hbm.at[w, r * G + i],
                             dstv.at[p * G + i], sem_id.at[p])

    def wait_didx(p):
        for i in range(G):
            pltpu.make_async_copy(dst_hbm.at[0, 0],
                                  dstv.at[p * G + i], sem_id.at[p]).wait()

    def fire_gather(p):
        for i in range(G):
            pltpu.async_copy(h_hbm.at[srcv.at[p * G + i]],
                             rows.at[p, i], sem_g.at[p])

    def wait_gather(p):
        for i in range(G):
            pltpu.make_async_copy(zrows_hbm.at[pl.ds(0, K)],
                                  rows.at[p, i], sem_g.at[p]).wait()

    def fire_scatter(p):
        for i in range(G):
            pltpu.async_copy(rows.at[p, i], acc_sh.at[dstv.at[p * G + i]],
                             sem_s.at[p], add=True)

    def wait_scatter(p):
        for i in range(G):
            pltpu.make_async_copy(zrows_hbm.at[pl.ds(0, K)],
                                  rows.at[p, i], sem_s.at[p]).wait()

    fire_sidx(0, 0)
    fire_sidx(1, 1)
    fire_didx(0, 0)
    wait_sidx(0)
    fire_gather(0)

    def body(r, carry):
        p = lax.rem(r, NB)
        pn = lax.rem(r + 1, NB)
        wait_gather(p)

        @pl.when(r + 1 >= NB)
        def _free_slot():
            wait_scatter(pn)

        wait_didx(p)
        fire_scatter(p)

        @pl.when(r + 1 < NR)
        def _next():
            wait_sidx(pn)
            fire_gather(pn)
            fire_didx(r + 1, pn)

            @pl.when(r + 2 < NR)
            def _next2():
                fire_sidx(r + 2, lax.rem(r + 2, NB))

        return carry

    lax.fori_loop(0, NR, body, 0, unroll=False)
    for d in range(1, NB):
        wait_scatter((NR - d) % NB)
    plsc.subcore_barrier()

    @pl.when(s < NZ)
    def _writeout():
        pltpu.sync_copy(acc_sh.at[pl.ds(s * ZR, ZR)],
                        agg_out.at[c, pl.ds(s * ZR, ZR)])


def _make_sc_agg(with_cnt):
    outs = jax.ShapeDtypeStruct((NC, N, D), jnp.float32)
    return pl.kernel(
        functools.partial(_sc_agg_body, with_cnt),
        out_type=(outs, outs) if with_cnt else (outs, outs),
        mesh=plsc.VectorSubcoreMesh(core_axis_name="c", subcore_axis_name="s",
                                    num_cores=NC, num_subcores=NS),
        scratch_types=[
            pltpu.VMEM((NB * G, K), jnp.int32),
            pltpu.VMEM((NB * G, K), jnp.int32),
            pltpu.VMEM((NB, G, K, D), jnp.float32),
            pltpu.VMEM_SHARED((N, D), jnp.float32),
            pltpu.SemaphoreType.DMA((NB,)),
            pltpu.SemaphoreType.DMA((NB,)),
            pltpu.SemaphoreType.DMA((NB,)),
            pltpu.SemaphoreType.DMA((NB,)),
            pltpu.SemaphoreType.DMA((2,)),
        ],
        name="sc_sage_agg_cnt" if with_cnt else "sc_sage_agg",
    )


_sc_agg_cnt = _make_sc_agg(True)
_sc_agg = _make_sc_agg(False)


def _tc_body(do_relu, aggp_ref, cntp_ref, h_ref, wl_ref, bl_ref, wr_ref,
             out_ref):
    agg = aggp_ref[0] + aggp_ref[1]
    cnt = cntp_ref[0] + cntp_ref[1]
    mean = agg / jnp.maximum(cnt, 1.0)
    dn = (((1,), (1,)), ((), ()))
    y = (lax.dot_general(mean, wl_ref[...], dn,
                         preferred_element_type=jnp.float32)
         + lax.dot_general(h_ref[...], wr_ref[...], dn,
                           preferred_element_type=jnp.float32)
         + bl_ref[...])
    out_ref[...] = jnp.maximum(y, 0.0) if do_relu else y


def _tc_layer(aggp, cntp, h, wl, bl, wr, do_relu):
    R = 1000
    grid = (N // R,)
    return pl.pallas_call(
        functools.partial(_tc_body, do_relu),
        grid=grid,
        in_specs=[
            pl.BlockSpec((NC, R, D), lambda i: (0, i, 0)),
            pl.BlockSpec((NC, R, D), lambda i: (0, i, 0)),
            pl.BlockSpec((R, D), lambda i: (i, 0)),
            pl.BlockSpec((D, D), lambda i: (0, 0)),
            pl.BlockSpec((1, D), lambda i: (0, 0)),
            pl.BlockSpec((D, D), lambda i: (0, 0)),
        ],
        out_specs=pl.BlockSpec((R, D), lambda i: (i, 0)),
        out_shape=jax.ShapeDtypeStruct((N, D), jnp.float32),
    )(aggp, cntp, h, wl, bl.reshape(1, D), wr)


def kernel(x, edge_index, Wl1, bl1, Wr1, Wl2, bl2, Wr2, Wl3, bl3, Wr3):
    src = edge_index[0].reshape(NW, NCHUNK, K)
    dst = edge_index[1].reshape(NW, NCHUNK, K)
    zrows = jnp.zeros((N, D), jnp.float32)
    ones = jnp.ones((K, D), jnp.float32)

    aggp, cntp = _sc_agg_cnt(x, src, dst, zrows, ones)
    h1 = _tc_layer(aggp, cntp, x, Wl1, bl1, Wr1, True)
    aggp, _ = _sc_agg(h1, src, dst, zrows, ones)
    h2 = _tc_layer(aggp, cntp, h1, Wl2, bl2, Wr2, True)
    aggp, _ = _sc_agg(h2, src, dst, zrows, ones)
    return _tc_layer(aggp, cntp, h2, Wl3, bl3, Wr3, False)
